# Initial kernel scaffold; baseline (speedup 1.0000x reference)
import jax
import jax.numpy as jnp
from jax.experimental import pallas as pl


def kernel(x, edge_index, edge_weight, enc_w, enc_b, conv_w, conv_b, res_w, res_b, dec_w, dec_b, gate_w, gate_b, weight_mlp, epsilons, lamda):
    raise NotImplementedError("write your pallas kernel here")


# speedup vs baseline: 5.8241x; 5.8241x over previous
import functools

import jax
import jax.numpy as jnp
from jax import lax
from jax.experimental import pallas as pl
from jax.experimental.pallas import tpu as pltpu
from jax.experimental.pallas import tpu_sc as plsc

N = 10000
E = 320000
NFEAT = 128
NHID = 64
NCLASS = 40
NLAYERS = 4

NTILES = 32
CHUNK = 128
CHUNKS_PT = 80
EPT = CHUNK * CHUNKS_PT
EPAD = NTILES * EPT
NPAD = 10240
DUMP = NPAD - 1
RPT = NPAD // 16
RB = 1024
GRID = NPAD // RB
VROWS = NTILES * CHUNKS_PT

_mesh = plsc.VectorSubcoreMesh(core_axis_name="c", subcore_axis_name="s")
_sc_params = pltpu.CompilerParams(use_tc_tiling_on_sc=False)


def _fori(n, body):
    lax.fori_loop(jnp.int32(0), jnp.int32(n), body, jnp.int32(0))


def _zero_acc_slice(buf, acc, s):
    w = buf.shape[1]
    zrow = jnp.zeros((16,), jnp.float32)

    def zb(r, _):
        for k in range(w // 16):
            buf[r, pl.ds(16 * k, 16)] = zrow
        return _

    _fori(CHUNK, zb)
    base = s * RPT
    for k in range(RPT // CHUNK):
        pltpu.sync_copy(buf, acc.at[pl.ds(base + k * CHUNK, CHUNK)])


@functools.partial(
    pl.kernel,
    mesh=_mesh,
    out_type=jax.ShapeDtypeStruct((2 * NPAD, 16), jnp.float32),
    scratch_types=[
        pltpu.VMEM((CHUNKS_PT, CHUNK), jnp.int32),
        pltpu.VMEM((CHUNKS_PT, CHUNK), jnp.int32),
        pltpu.VMEM((CHUNK, 16), jnp.float32),
        pltpu.VMEM((CHUNK, 16), jnp.float32),
        pltpu.VMEM_SHARED((NPAD, 16), jnp.float32),
    ],
    compiler_params=_sc_params,
)
def _sc_deg(col_hbm, colself_hbm, dg_out, colv, selfv, c0buf, c1buf, acc):
    c = lax.axis_index("c")
    s = lax.axis_index("s")
    t = c * 16 + s
    pltpu.sync_copy(col_hbm.at[pl.ds(t * CHUNKS_PT, CHUNKS_PT)], colv)
    pltpu.sync_copy(colself_hbm.at[pl.ds(t * CHUNKS_PT, CHUNKS_PT)], selfv)

    _zero_acc_slice(c0buf, acc, s)

    lanef = lax.iota(jnp.int32, 16).astype(jnp.float32)
    e0 = jnp.maximum(1.0 - lanef, 0.0)
    e1 = jnp.maximum(1.0 - jnp.abs(lanef - 1.0), 0.0)

    @pl.loop(jnp.int32(0), jnp.int32(CHUNK))
    def fill(r):
        c0buf[r, pl.ds(0, 16)] = e0
        c1buf[r, pl.ds(0, 16)] = e1

    plsc.subcore_barrier()

    @pl.loop(jnp.int32(0), jnp.int32(CHUNKS_PT))
    def chunk_body(j):
        pltpu.sync_copy(c0buf, acc.at[colv.at[j]], add=True)
        pltpu.sync_copy(c1buf, acc.at[selfv.at[j]], add=True)

    plsc.subcore_barrier()
    base = s * RPT
    pltpu.sync_copy(acc.at[pl.ds(base, RPT)],
                    dg_out.at[pl.ds(c * NPAD + base, RPT)])


@functools.partial(
    pl.kernel,
    mesh=_mesh,
    out_type=jax.ShapeDtypeStruct((2 * NPAD, NHID), jnp.float32),
    scratch_types=[
        pltpu.VMEM((CHUNKS_PT, CHUNK), jnp.int32),
        pltpu.VMEM((CHUNKS_PT, CHUNK), jnp.int32),
        pltpu.VMEM((CHUNK, NHID), jnp.float32),
        pltpu.VMEM((CHUNK, NHID), jnp.float32),
        pltpu.VMEM((CHUNK, NHID), jnp.float32),
        pltpu.VMEM((CHUNK, NHID), jnp.float32),
        pltpu.VMEM_SHARED((NPAD, NHID), jnp.float32),
        pltpu.SemaphoreType.DMA,
        pltpu.SemaphoreType.DMA,
        pltpu.SemaphoreType.DMA,
    ],
    compiler_params=_sc_params,
)
def _sc_ax3(row_hbm, col_hbm, xw_hbm, x_hbm, acc_out,
            rowv, colv, bA, bB, bC, bO, acc, sem1, sem2, sem3):
    c = lax.axis_index("c")
    s = lax.axis_index("s")
    t = c * 16 + s
    pltpu.sync_copy(row_hbm.at[pl.ds(t * CHUNKS_PT, CHUNKS_PT)], rowv)
    pltpu.sync_copy(col_hbm.at[pl.ds(t * CHUNKS_PT, CHUNKS_PT)], colv)

    _zero_acc_slice(bO, acc, s)
    plsc.subcore_barrier()

    @pl.loop(jnp.int32(0), jnp.int32(CHUNKS_PT))
    def chunk_body(j):
        ridx = rowv.at[j]
        cidx = colv.at[j]
        d1 = pltpu.async_copy(xw_hbm.at[ridx], bA, sem1)
        d2 = pltpu.async_copy(xw_hbm.at[cidx], bB, sem2)
        d3 = pltpu.async_copy(x_hbm.at[cidx], bC, sem3)
        d1.wait()
        d2.wait()
        d3.wait()

        @pl.loop(jnp.int32(0), jnp.int32(CHUNK))
        def row_body(r):
            for k in range(NHID // 16):
                sl = pl.ds(16 * k, 16)
                d = bA[r, sl] - bB[r, sl]
                d = jnp.clip(d, -12.0, 12.0)
                u = jnp.exp(d + d)
                th = (u - 1.0) / (u + 1.0)
                bO[r, sl] = th * bC[r, sl]

        pltpu.sync_copy(bO, acc.at[cidx], add=True)

    plsc.subcore_barrier()
    base = s * RPT
    pltpu.sync_copy(acc.at[pl.ds(base, RPT)],
                    acc_out.at[pl.ds(c * NPAD + base, RPT)])


@functools.partial(
    pl.kernel,
    mesh=_mesh,
    out_type=jax.ShapeDtypeStruct((2 * NPAD, NHID), jnp.float32),
    scratch_types=[
        pltpu.VMEM((CHUNKS_PT, CHUNK), jnp.int32),
        pltpu.VMEM((CHUNKS_PT, CHUNK), jnp.int32),
        pltpu.VMEM((CHUNK, NHID), jnp.float32),
        pltpu.VMEM_SHARED((NPAD, NHID), jnp.float32),
        pltpu.SemaphoreType.DMA,
    ],
    compiler_params=_sc_params,
)
def _sc_conv(row_hbm, col_hbm, xwd_hbm, acc_out,
             rowv, colv, bD, acc, sem1):
    c = lax.axis_index("c")
    s = lax.axis_index("s")
    t = c * 16 + s
    pltpu.sync_copy(row_hbm.at[pl.ds(t * CHUNKS_PT, CHUNKS_PT)], rowv)
    pltpu.sync_copy(col_hbm.at[pl.ds(t * CHUNKS_PT, CHUNKS_PT)], colv)

    _zero_acc_slice(bD, acc, s)
    plsc.subcore_barrier()

    @pl.loop(jnp.int32(0), jnp.int32(CHUNKS_PT))
    def chunk_body(j):
        pltpu.async_copy(xwd_hbm.at[rowv.at[j]], bD, sem1).wait()
        pltpu.sync_copy(bD, acc.at[colv.at[j]], add=True)

    plsc.subcore_barrier()
    base = s * RPT
    pltpu.sync_copy(acc.at[pl.ds(base, RPT)],
                    acc_out.at[pl.ds(c * NPAD + base, RPT)])


_I0 = None


def _z(i):
    return jnp.zeros_like(i)


def _full(shape):
    n = len(shape)
    return pl.BlockSpec(shape, lambda i: (_z(i),) * n)


def _rows(width):
    return pl.BlockSpec((RB, width), lambda i: (i, _z(i)))


def _rows3(width):
    return pl.BlockSpec((2, RB, width), lambda i: (_z(i), i, _z(i)))


def _tc_val_body(row_ref, col_ref, out):
    rv = row_ref[...]
    cv = col_ref[...]
    out[...] = jnp.where(rv == cv, cv, jnp.int32(DUMP))


_tc_val = pl.pallas_call(
    _tc_val_body,
    grid=(4,),
    in_specs=[
        pl.BlockSpec((VROWS // 4, CHUNK), lambda i: (i, _z(i))),
        pl.BlockSpec((VROWS // 4, CHUNK), lambda i: (i, _z(i))),
    ],
    out_specs=pl.BlockSpec((VROWS // 4, CHUNK), lambda i: (i, _z(i))),
    out_shape=jax.ShapeDtypeStruct((VROWS, CHUNK), jnp.int32),
)


def _tc_prep_body(xp_ref, encw, encb, dg_ref, convw, wmlp,
                  x_o, xw_o, xwt_o, xwd_o, dinvT_o, bT_o):
    X = jax.nn.relu(
        jnp.dot(xp_ref[...], encw[...], preferred_element_type=jnp.float32)
        + encb[...])
    dg = dg_ref[...]
    allcnt = dg[0, :, 0:1] + dg[1, :, 0:1]
    selfc = dg[0, :, 1:2] + dg[1, :, 1:2]
    deg = 1.0 + allcnt - selfc
    dinv = lax.rsqrt(deg)
    b = dinv * dinv * (1.0 - selfc)
    xw = jnp.dot(X, convw[...], preferred_element_type=jnp.float32)
    x_o[...] = X
    xw_o[...] = xw
    xwt_o[...] = jnp.dot(X, wmlp[...], preferred_element_type=jnp.float32)
    xwd_o[...] = dinv * xw
    dinvT_o[...] = jnp.broadcast_to(dinv, (RB, NHID))
    bT_o[...] = jnp.broadcast_to(b, (RB, NHID))


_tc_prep = pl.pallas_call(
    _tc_prep_body,
    grid=(GRID,),
    in_specs=[
        _rows(NFEAT), _full((NFEAT, NHID)), _full((1, NHID)),
        _rows3(16), _full((NHID, NHID)), _full((NHID, NHID)),
    ],
    out_specs=[_rows(NHID)] * 6,
    out_shape=[jax.ShapeDtypeStruct((NPAD, NHID), jnp.float32)] * 6,
)


def _tc_update_body(x_ref, xw_ref, acc3_ref, accc_ref, dinvT, bT,
                    convb, resw, resb, wmlp, convw, lam, eps,
                    x_o, xw_o, xwt_o, xwd_o):
    X = x_ref[...]
    xw = xw_ref[...]
    acc3 = acc3_ref[...]
    accc = accc_ref[...]
    ax3 = acc3[0] + acc3[1]
    accs = accc[0] + accc[1]
    conv_out = dinvT[...] * accs + bT[...] * xw + convb[...]
    res = jnp.dot(xw, resw[...], preferred_element_type=jnp.float32) + resb[...]
    ax2 = jax.nn.relu(conv_out - res)
    coefl = jnp.tanh(lam[...])
    coefe = 1.0 + jnp.tanh(eps[...])
    Xn = X * coefe + (ax2 + coefl * ax3 - X)
    xwn = jnp.dot(Xn, convw[...], preferred_element_type=jnp.float32)
    x_o[...] = Xn
    xw_o[...] = xwn
    xwt_o[...] = jnp.dot(Xn, wmlp[...], preferred_element_type=jnp.float32)
    xwd_o[...] = dinvT[...] * xwn


_tc_update = pl.pallas_call(
    _tc_update_body,
    grid=(GRID,),
    in_specs=[
        _rows(NHID), _rows(NHID), _rows3(NHID), _rows3(NHID),
        _rows(NHID), _rows(NHID),
        _full((1, NHID)), _full((NHID, NHID)), _full((1, NHID)),
        _full((NHID, NHID)), _full((NHID, NHID)),
        _full((1, NHID)), _full((1, NHID)),
    ],
    out_specs=[_rows(NHID)] * 4,
    out_shape=[jax.ShapeDtypeStruct((NPAD, NHID), jnp.float32)] * 4,
)


def _tc_dec_body(x_ref, decw, decb, out):
    out[...] = (
        jnp.dot(x_ref[...], decw[...], preferred_element_type=jnp.float32)
        + decb[...])


_tc_dec = pl.pallas_call(
    _tc_dec_body,
    grid=(GRID,),
    in_specs=[_rows(NHID), _full((NHID, 128)), _full((1, 128))],
    out_specs=_rows(128),
    out_shape=jax.ShapeDtypeStruct((NPAD, 128), jnp.float32),
)


def kernel(x, edge_index, edge_weight, enc_w, enc_b, conv_w, conv_b,
           res_w, res_b, dec_w, dec_b, gate_w, gate_b, weight_mlp,
           epsilons, lamda):
    del edge_weight, gate_w, gate_b
    row = edge_index[0].astype(jnp.int32)
    col = edge_index[1].astype(jnp.int32)
    padlen = EPAD - E
    padv = jnp.full((padlen,), N, jnp.int32)
    row2 = jnp.concatenate([row, padv]).reshape(VROWS, CHUNK)
    col2 = jnp.concatenate([col, padv]).reshape(VROWS, CHUNK)

    xp = jnp.pad(x, ((0, NPAD - N), (0, 0)))
    encb = enc_b.reshape(1, NHID)
    convb = conv_b.reshape(1, NHID)
    resb = res_b.reshape(1, NHID)
    decw = jnp.pad(dec_w, ((0, 0), (0, 128 - NCLASS)))
    decb = jnp.pad(dec_b, (0, 128 - NCLASS)).reshape(1, 128)

    colself = _tc_val(row2, col2)
    dg = _sc_deg(col2, colself).reshape(2, NPAD, 16)
    X, xw, XW, XWD, dinvT, bT = _tc_prep(
        xp, enc_w, encb, dg, conv_w, weight_mlp)

    for i in range(NLAYERS):
        acc3 = _sc_ax3(row2, col2, XW, X).reshape(2, NPAD, NHID)
        accc = _sc_conv(row2, col2, XWD).reshape(2, NPAD, NHID)
        lam = lamda[i].reshape(1, NHID)
        eps = epsilons[i].reshape(1, NHID)
        X, xw, XW, XWD = _tc_update(
            X, xw, acc3, accc, dinvT, bT,
            convb, res_w, resb, weight_mlp, conv_w, lam, eps)

    out = _tc_dec(X, decw, decb)
    return out[:N, :NCLASS]
